# baseline (speedup 1.0000x reference)
import functools

import jax
import jax.numpy as jnp
from jax import lax
from jax.experimental import pallas as pl
from jax.experimental.pallas import tpu as pltpu
from jax.experimental.pallas import tpu_sc as plsc

N = 10000
E = 320000
D = 128
H = 128
G = 256
NCLS = 10

NSC = 2
NTILE = 16
NW = NSC * NTILE
NPAD = 10240
RPT = NPAD // NTILE

EPW = E // NW

CHD = 80
NCHD = EPW // CHD

CH = 80
NCHUNK = EPW // CH


def _deg_body(dst_hbm, out_hbm, idx0, idx1, ones_v, stage_v, i0, i1, table):
    idx = (idx0, idx1)
    isem = (i0, i1)
    c = lax.axis_index("c")
    s = lax.axis_index("s")
    w = c * NTILE + s

    def _ones(i, _):
        ones_v[pl.ds(i * 16, 16)] = jnp.ones((16,), jnp.float32)
        return 0

    lax.fori_loop(0, CHD // 16, _ones, 0)

    def _zero(i, _):
        stage_v[pl.ds(i * 16, 16)] = jnp.zeros((16,), jnp.float32)
        return 0

    lax.fori_loop(0, RPT // 16, _zero, 0)
    pltpu.sync_copy(stage_v, table.at[pl.ds(s * RPT, RPT)])
    plsc.subcore_barrier()

    def _chunk(j, b, early, pref):
        if not early:
            pltpu.make_async_copy(dst_hbm.at[pl.ds(0, CHD)], idx[b],
                                  isem[b]).wait()
        pltpu.sync_copy(ones_v, table.at[idx[b]], add=True)
        if pref:
            pltpu.async_copy(dst_hbm.at[pl.ds(w * EPW + (j + 2) * CHD,
                                              CHD)], idx[b], isem[b])

    for j in range(2):
        pltpu.sync_copy(dst_hbm.at[pl.ds(w * EPW + j * CHD, CHD)], idx[j])
    _chunk(0, 0, True, True)
    _chunk(1, 1, True, True)

    def _body(g, _):
        j0 = 2 * g + 2
        _chunk(j0, 0, False, True)
        _chunk(j0 + 1, 1, False, True)
        return 0

    lax.fori_loop(0, (NCHD - 5) // 2, _body, 0)
    _chunk(NCHD - 3, 0, False, True)
    _chunk(NCHD - 2, 1, False, False)
    _chunk(NCHD - 1, 0, False, False)
    plsc.subcore_barrier()
    pltpu.sync_copy(table.at[pl.ds(s * RPT, RPT)],
                    out_hbm.at[c, pl.ds(s * RPT, RPT)])


def _agg_body(u_hbm, src_hbm, dst_hbm, out_hbm, sidx0, sidx1, didx0, didx1,
              rows0, rows1, gs0, gs1, iss0, iss1, isd0, isd1, ss0, ss1,
              table):
    rows = (rows0, rows1)
    sidx = (sidx0, sidx1)
    didx = (didx0, didx1)
    gsem = (gs0, gs1)
    isem_s = (iss0, iss1)
    isem_d = (isd0, isd1)
    ssem = (ss0, ss1)
    c = lax.axis_index("c")
    s = lax.axis_index("s")
    w = c * NTILE + s

    def _zero(i, _):
        rows0[i // 8, pl.ds((i % 8) * 16, 16)] = jnp.zeros((16,),
                                                           jnp.float32)
        return 0

    lax.fori_loop(0, CH * (H // 16), _zero, 0)

    def _ztab(k, _):
        pltpu.sync_copy(rows0, table.at[pl.ds(s * RPT + k * CH, CH)])
        return 0

    lax.fori_loop(0, RPT // CH, _ztab, 0)
    plsc.subcore_barrier()

    def _wait_g(b):
        pltpu.make_async_copy(u_hbm.at[sidx[b]], rows[b], gsem[b]).wait()

    def _wait_is(p):
        pltpu.make_async_copy(src_hbm.at[pl.ds(0, CH)], sidx[p],
                              isem_s[p]).wait()

    def _wait_id(p):
        pltpu.make_async_copy(dst_hbm.at[pl.ds(0, CH)], didx[p],
                              isem_d[p]).wait()

    def _wait_s(b):
        pltpu.make_async_copy(rows[b], table.at[didx[b]], ssem[b]).wait()

    def _chunk(j, b, fi, sec, last, pref):
        _wait_g(b)
        if not last:
            if not fi:
                _wait_is(1 - b)
                _wait_s(1 - b)
                pltpu.async_copy(
                    dst_hbm.at[pl.ds(w * EPW + (j + 1) * CH, CH)],
                    didx[1 - b], isem_d[1 - b])
            pltpu.async_copy(u_hbm.at[sidx[1 - b]], rows[1 - b],
                             gsem[1 - b])
        if not (fi or sec):
            _wait_id(b)
        pltpu.async_copy(rows[b], table.at[didx[b]], ssem[b], add=True)
        if pref:
            pltpu.async_copy(
                src_hbm.at[pl.ds(w * EPW + (j + 2) * CH, CH)],
                sidx[b], isem_s[b])

    for j in range(2):
        pltpu.sync_copy(src_hbm.at[pl.ds(w * EPW + j * CH, CH)], sidx[j])
        pltpu.sync_copy(dst_hbm.at[pl.ds(w * EPW + j * CH, CH)], didx[j])
    pltpu.async_copy(u_hbm.at[sidx0], rows0, gsem[0])
    _chunk(0, 0, True, False, False, True)
    _chunk(1, 1, False, True, False, True)

    def _body(g, _):
        j0 = 2 * g + 2
        _chunk(j0, 0, False, False, False, True)
        _chunk(j0 + 1, 1, False, False, False, True)
        return 0

    lax.fori_loop(0, (NCHUNK - 5) // 2, _body, 0)
    _chunk(NCHUNK - 3, 0, False, False, False, True)
    _chunk(NCHUNK - 2, 1, False, False, False, False)
    _chunk(NCHUNK - 1, 0, False, False, True, False)
    _wait_s(1)
    _wait_s(0)
    plsc.subcore_barrier()
    pltpu.sync_copy(table.at[pl.ds(s * RPT, RPT)],
                    out_hbm.at[c, pl.ds(s * RPT, RPT)])


@functools.cache
def _sc_kernels():
    mesh = plsc.VectorSubcoreMesh(core_axis_name="c", subcore_axis_name="s")
    deg = pl.kernel(
        _deg_body,
        out_type=jax.ShapeDtypeStruct((NSC, NPAD), jnp.float32),
        mesh=mesh,
        scratch_types=[
            pltpu.VMEM((CHD,), jnp.int32),
            pltpu.VMEM((CHD,), jnp.int32),
            pltpu.VMEM((CHD,), jnp.float32),
            pltpu.VMEM((RPT,), jnp.float32),
            pltpu.SemaphoreType.DMA,
            pltpu.SemaphoreType.DMA,
            pltpu.VMEM_SHARED((NPAD,), jnp.float32),
        ],
    )
    agg = pl.kernel(
        _agg_body,
        out_type=jax.ShapeDtypeStruct((NSC, NPAD, H), jnp.float32),
        mesh=mesh,
        scratch_types=[
            pltpu.VMEM((CH,), jnp.int32),
            pltpu.VMEM((CH,), jnp.int32),
            pltpu.VMEM((CH,), jnp.int32),
            pltpu.VMEM((CH,), jnp.int32),
            pltpu.VMEM((CH, H), jnp.float32),
            pltpu.VMEM((CH, H), jnp.float32),
            pltpu.SemaphoreType.DMA,
            pltpu.SemaphoreType.DMA,
            pltpu.SemaphoreType.DMA,
            pltpu.SemaphoreType.DMA,
            pltpu.SemaphoreType.DMA,
            pltpu.SemaphoreType.DMA,
            pltpu.SemaphoreType.DMA,
            pltpu.SemaphoreType.DMA,
            pltpu.VMEM_SHARED((NPAD, H), jnp.float32),
        ],
    )
    return deg, agg


_BLK = 2000
_NBLK = N // _BLK


def _tc1a_body(x_ref, w1_ref, h_ref):
    h_ref[...] = jnp.dot(x_ref[...], w1_ref[...],
                         preferred_element_type=jnp.float32)


def _tc1b_body(h_ref, deg_ref, u_ref, dinv_ref):
    deg = deg_ref[0] + deg_ref[1] + 1.0
    dinv = lax.rsqrt(deg)
    dinv_ref[...] = dinv
    u_ref[...] = h_ref[...] * dinv


def _tc2_body(s_ref, u1_ref, dinv_ref, b1_ref, w2_ref, u2_ref):
    dinv = dinv_ref[...]
    h = jax.nn.relu(dinv * (s_ref[0] + s_ref[1] + u1_ref[...]) + b1_ref[...])
    u2_ref[...] = jnp.dot(h, w2_ref[...],
                          preferred_element_type=jnp.float32) * dinv


def _tc3_body(s_ref, u2_ref, dinv_ref, b2_ref, batch_ref,
              t0a_ref, t0b_ref, c0b1_ref, c0b2_ref,
              t1a_ref, t1b_ref, c1b1_ref, c1b2_ref,
              linw_ref, linb_ref, out_ref, sums, cnt):
    i = pl.program_id(0)

    @pl.when(i == 0)
    def _():
        sums[...] = jnp.zeros_like(sums)
        cnt[...] = jnp.zeros_like(cnt)

    h = jax.nn.relu(dinv_ref[...] * (s_ref[0] + s_ref[1] + u2_ref[...])
                    + b2_ref[...])
    gids = lax.broadcasted_iota(jnp.int32, (_BLK, G), 1)
    onehot = (batch_ref[...] == gids).astype(jnp.float32)
    dn = (((0,), (0,)), ((), ()))
    sums[...] += lax.dot_general(onehot, h, dn,
                                 preferred_element_type=jnp.float32)
    cnt[...] += lax.dot_general(onehot, jnp.ones((_BLK, 1), jnp.float32), dn,
                                preferred_element_type=jnp.float32)

    @pl.when(i == _NBLK - 1)
    def _():
        pooled = sums[...] / jnp.maximum(cnt[...], 1.0)
        a = jax.nn.relu(jnp.dot(pooled, t0a_ref[...],
                                preferred_element_type=jnp.float32)
                        + c0b1_ref[...])
        a = jax.nn.relu(jnp.dot(a, t0b_ref[...],
                                preferred_element_type=jnp.float32)
                        + c0b2_ref[...])
        t = jax.nn.relu(a + pooled)
        b = jax.nn.relu(jnp.dot(t, t1a_ref[...],
                                preferred_element_type=jnp.float32)
                        + c1b1_ref[...])
        b = jax.nn.relu(jnp.dot(b, t1b_ref[...],
                                preferred_element_type=jnp.float32)
                        + c1b2_ref[...])
        t2 = jax.nn.relu(b + t)
        out_ref[...] = (jnp.dot(t2, linw_ref[...],
                                preferred_element_type=jnp.float32)
                        + linb_ref[...])


def _row_spec(last):
    return pl.BlockSpec((_BLK, last), lambda i: (i, 0))


def _full_spec(shape):
    nd = len(shape)
    return pl.BlockSpec(shape, lambda i: (0,) * nd)


def _sc_spec(last):
    return pl.BlockSpec((NSC, _BLK, last), lambda i: (0, i, 0))


def kernel(x, edge_index, batch, W1, b1, W2, b2,
           c0w1, c0b1, c0w2, c0b2, c1w1, c1b1, c1w2, c1b2, lin_w, lin_b):
    src = edge_index[0]
    dst = edge_index[1]
    _deg_kernel, _agg_kernel = _sc_kernels()

    degp = _deg_kernel(dst).reshape(NSC, NPAD, 1)

    h1 = pl.pallas_call(
        _tc1a_body,
        grid=(_NBLK,),
        in_specs=[_row_spec(D), _full_spec((D, H))],
        out_specs=_row_spec(H),
        out_shape=jax.ShapeDtypeStruct((N, H), jnp.float32),
    )(x, W1)

    u1, dinv = pl.pallas_call(
        _tc1b_body,
        grid=(_NBLK,),
        in_specs=[_row_spec(H), _sc_spec(1)],
        out_specs=[_row_spec(H), _row_spec(1)],
        out_shape=[jax.ShapeDtypeStruct((N, H), jnp.float32),
                   jax.ShapeDtypeStruct((N, 1), jnp.float32)],
    )(h1, degp)

    s1 = _agg_kernel(u1, src, dst)

    u2 = pl.pallas_call(
        _tc2_body,
        grid=(_NBLK,),
        in_specs=[_sc_spec(H), _row_spec(H), _row_spec(1),
                  _full_spec((1, H)), _full_spec((H, H))],
        out_specs=_row_spec(H),
        out_shape=jax.ShapeDtypeStruct((N, H), jnp.float32),
    )(s1, u1, dinv, b1.reshape(1, H), W2)

    s2 = _agg_kernel(u2, src, dst)

    out = pl.pallas_call(
        _tc3_body,
        grid=(_NBLK,),
        in_specs=[_sc_spec(H), _row_spec(H), _row_spec(1),
                  _full_spec((1, H)), _row_spec(1),
                  _full_spec((H, H)), _full_spec((H, H)),
                  _full_spec((1, H)), _full_spec((1, H)),
                  _full_spec((H, H)), _full_spec((H, H)),
                  _full_spec((1, H)), _full_spec((1, H)),
                  _full_spec((H, NCLS)), _full_spec((1, NCLS))],
        out_specs=_full_spec((G, NCLS)),
        out_shape=jax.ShapeDtypeStruct((G, NCLS), jnp.float32),
        scratch_shapes=[pltpu.VMEM((G, H), jnp.float32),
                        pltpu.VMEM((G, 1), jnp.float32)],
    )(s2, u2, dinv, b2.reshape(1, H), batch.reshape(N, 1),
      c0w1[:, :, -1].T, c0w2[:, :, -1].T,
      c0b1.reshape(1, H), c0b2.reshape(1, H),
      c1w1[:, :, -1].T, c1w2[:, :, -1].T,
      c1b1.reshape(1, H), c1b2.reshape(1, H),
      lin_w, lin_b.reshape(1, NCLS))

    return out


# speedup vs baseline: 1.0041x; 1.0041x over previous
import functools

import jax
import jax.numpy as jnp
from jax import lax
from jax.experimental import pallas as pl
from jax.experimental.pallas import tpu as pltpu
from jax.experimental.pallas import tpu_sc as plsc

N = 10000
E = 320000
D = 128
H = 128
G = 256
NCLS = 10

NSC = 2
NTILE = 16
NW = NSC * NTILE
NPAD = 10240
RPT = NPAD // NTILE

EPW = E // NW

CHD = 80
NCHD = EPW // CHD

CH = 80
NCHUNK = EPW // CH


def _deg_body(dst_hbm, out_hbm, idx0, idx1, ones_v, stage_v, i0, i1, table):
    idx = (idx0, idx1)
    isem = (i0, i1)
    c = lax.axis_index("c")
    s = lax.axis_index("s")
    w = c * NTILE + s

    def _ones(i, _):
        ones_v[pl.ds(i * 16, 16)] = jnp.ones((16,), jnp.float32)
        return 0

    lax.fori_loop(0, CHD // 16, _ones, 0)

    def _zero(i, _):
        stage_v[pl.ds(i * 16, 16)] = jnp.zeros((16,), jnp.float32)
        return 0

    lax.fori_loop(0, RPT // 16, _zero, 0)
    pltpu.sync_copy(stage_v, table.at[pl.ds(s * RPT, RPT)])
    plsc.subcore_barrier()

    def _chunk(j, b, early, pref):
        if not early:
            pltpu.make_async_copy(dst_hbm.at[pl.ds(0, CHD)], idx[b],
                                  isem[b]).wait()
        pltpu.sync_copy(ones_v, table.at[idx[b]], add=True)
        if pref:
            pltpu.async_copy(dst_hbm.at[pl.ds(w * EPW + (j + 2) * CHD,
                                              CHD)], idx[b], isem[b])

    for j in range(2):
        pltpu.sync_copy(dst_hbm.at[pl.ds(w * EPW + j * CHD, CHD)], idx[j])
    _chunk(0, 0, True, True)
    _chunk(1, 1, True, True)

    def _body(g, _):
        j0 = 2 * g + 2
        _chunk(j0, 0, False, True)
        _chunk(j0 + 1, 1, False, True)
        return 0

    lax.fori_loop(0, (NCHD - 5) // 2, _body, 0)
    _chunk(NCHD - 3, 0, False, True)
    _chunk(NCHD - 2, 1, False, False)
    _chunk(NCHD - 1, 0, False, False)
    plsc.subcore_barrier()
    pltpu.sync_copy(table.at[pl.ds(s * RPT, RPT)],
                    out_hbm.at[c, pl.ds(s * RPT, RPT)])


def _agg_body(u_hbm, src_hbm, dst_hbm, out_hbm, sidx0, sidx1, didx0, didx1,
              rows0, rows1, gs0, gs1, iss0, iss1, isd0, isd1, ss0, ss1,
              table):
    rows = (rows0, rows1)
    sidx = (sidx0, sidx1)
    didx = (didx0, didx1)
    gsem = (gs0, gs1)
    isem_s = (iss0, iss1)
    isem_d = (isd0, isd1)
    ssem = (ss0, ss1)
    c = lax.axis_index("c")
    s = lax.axis_index("s")
    w = c * NTILE + s

    def _zero(i, _):
        rows0[i // 8, pl.ds((i % 8) * 16, 16)] = jnp.zeros((16,),
                                                           jnp.float32)
        return 0

    lax.fori_loop(0, CH * (H // 16), _zero, 0)

    def _ztab(k, _):
        pltpu.sync_copy(rows0, table.at[pl.ds(s * RPT + k * CH, CH)])
        return 0

    lax.fori_loop(0, RPT // CH, _ztab, 0)
    plsc.subcore_barrier()

    def _wait_g(b):
        pltpu.make_async_copy(u_hbm.at[sidx[b]], rows[b], gsem[b]).wait()

    def _wait_is(p):
        pltpu.make_async_copy(src_hbm.at[pl.ds(0, CH)], sidx[p],
                              isem_s[p]).wait()

    def _wait_id(p):
        pltpu.make_async_copy(dst_hbm.at[pl.ds(0, CH)], didx[p],
                              isem_d[p]).wait()

    def _wait_s(b):
        pltpu.make_async_copy(rows[b], table.at[didx[b]], ssem[b]).wait()

    def _chunk(j, b, fi, sec, last, pref):
        _wait_g(b)
        if not last:
            if not fi:
                _wait_is(1 - b)
                _wait_s(1 - b)
                pltpu.async_copy(
                    dst_hbm.at[pl.ds(w * EPW + (j + 1) * CH, CH)],
                    didx[1 - b], isem_d[1 - b])
            pltpu.async_copy(u_hbm.at[sidx[1 - b]], rows[1 - b],
                             gsem[1 - b])
        if not (fi or sec):
            _wait_id(b)
        pltpu.async_copy(rows[b], table.at[didx[b]], ssem[b], add=True)
        if pref:
            pltpu.async_copy(
                src_hbm.at[pl.ds(w * EPW + (j + 2) * CH, CH)],
                sidx[b], isem_s[b])

    for j in range(2):
        pltpu.sync_copy(src_hbm.at[pl.ds(w * EPW + j * CH, CH)], sidx[j])
        pltpu.sync_copy(dst_hbm.at[pl.ds(w * EPW + j * CH, CH)], didx[j])
    pltpu.async_copy(u_hbm.at[sidx0], rows0, gsem[0])
    _chunk(0, 0, True, False, False, True)
    _chunk(1, 1, False, True, False, True)

    def _body(g, _):
        j0 = 2 * g + 2
        _chunk(j0, 0, False, False, False, True)
        _chunk(j0 + 1, 1, False, False, False, True)
        return 0

    lax.fori_loop(0, (NCHUNK - 5) // 2, _body, 0)
    _chunk(NCHUNK - 3, 0, False, False, False, True)
    _chunk(NCHUNK - 2, 1, False, False, False, False)
    _chunk(NCHUNK - 1, 0, False, False, True, False)
    _wait_s(1)
    _wait_s(0)
    plsc.subcore_barrier()
    pltpu.sync_copy(table.at[pl.ds(s * RPT, RPT)],
                    out_hbm.at[c, pl.ds(s * RPT, RPT)])


@functools.cache
def _sc_kernels():
    mesh = plsc.VectorSubcoreMesh(core_axis_name="c", subcore_axis_name="s")
    deg = pl.kernel(
        _deg_body,
        out_type=jax.ShapeDtypeStruct((NSC, NPAD), jnp.float32),
        mesh=mesh,
        scratch_types=[
            pltpu.VMEM((CHD,), jnp.int32),
            pltpu.VMEM((CHD,), jnp.int32),
            pltpu.VMEM((CHD,), jnp.float32),
            pltpu.VMEM((RPT,), jnp.float32),
            pltpu.SemaphoreType.DMA,
            pltpu.SemaphoreType.DMA,
            pltpu.VMEM_SHARED((NPAD,), jnp.float32),
        ],
    )
    agg = pl.kernel(
        _agg_body,
        out_type=jax.ShapeDtypeStruct((NSC, NPAD, H), jnp.float32),
        mesh=mesh,
        scratch_types=[
            pltpu.VMEM((CH,), jnp.int32),
            pltpu.VMEM((CH,), jnp.int32),
            pltpu.VMEM((CH,), jnp.int32),
            pltpu.VMEM((CH,), jnp.int32),
            pltpu.VMEM((CH, H), jnp.float32),
            pltpu.VMEM((CH, H), jnp.float32),
            pltpu.SemaphoreType.DMA,
            pltpu.SemaphoreType.DMA,
            pltpu.SemaphoreType.DMA,
            pltpu.SemaphoreType.DMA,
            pltpu.SemaphoreType.DMA,
            pltpu.SemaphoreType.DMA,
            pltpu.SemaphoreType.DMA,
            pltpu.SemaphoreType.DMA,
            pltpu.VMEM_SHARED((NPAD, H), jnp.float32),
        ],
    )
    return deg, agg


_BLK = 2000
_NBLK = N // _BLK


def _tc1_body(x_ref, w1_ref, deg_ref, u_ref, dinv_ref):
    deg = deg_ref[0] + deg_ref[1] + 1.0
    dinv = lax.rsqrt(deg)
    dinv_ref[...] = dinv
    u_ref[...] = jnp.dot(x_ref[...], w1_ref[...],
                         preferred_element_type=jnp.float32) * dinv


def _tc2_body(s_ref, u1_ref, dinv_ref, b1_ref, w2_ref, u2_ref):
    dinv = dinv_ref[...]
    h = jax.nn.relu(dinv * (s_ref[0] + s_ref[1] + u1_ref[...]) + b1_ref[...])
    u2_ref[...] = jnp.dot(h, w2_ref[...],
                          preferred_element_type=jnp.float32) * dinv


def _tc3_body(s_ref, u2_ref, dinv_ref, b2_ref, batch_ref,
              t0a_ref, t0b_ref, c0b1_ref, c0b2_ref,
              t1a_ref, t1b_ref, c1b1_ref, c1b2_ref,
              linw_ref, linb_ref, out_ref, sums, cnt):
    i = pl.program_id(0)

    @pl.when(i == 0)
    def _():
        sums[...] = jnp.zeros_like(sums)
        cnt[...] = jnp.zeros_like(cnt)

    h = jax.nn.relu(dinv_ref[...] * (s_ref[0] + s_ref[1] + u2_ref[...])
                    + b2_ref[...])
    gids = lax.broadcasted_iota(jnp.int32, (_BLK, G), 1)
    onehot = (batch_ref[...] == gids).astype(jnp.float32)
    dn = (((0,), (0,)), ((), ()))
    sums[...] += lax.dot_general(onehot, h, dn,
                                 preferred_element_type=jnp.float32)
    cnt[...] += lax.dot_general(onehot, jnp.ones((_BLK, 1), jnp.float32), dn,
                                preferred_element_type=jnp.float32)

    @pl.when(i == _NBLK - 1)
    def _():
        pooled = sums[...] / jnp.maximum(cnt[...], 1.0)
        a = jax.nn.relu(jnp.dot(pooled, t0a_ref[...],
                                preferred_element_type=jnp.float32)
                        + c0b1_ref[...])
        a = jax.nn.relu(jnp.dot(a, t0b_ref[...],
                                preferred_element_type=jnp.float32)
                        + c0b2_ref[...])
        t = jax.nn.relu(a + pooled)
        b = jax.nn.relu(jnp.dot(t, t1a_ref[...],
                                preferred_element_type=jnp.float32)
                        + c1b1_ref[...])
        b = jax.nn.relu(jnp.dot(b, t1b_ref[...],
                                preferred_element_type=jnp.float32)
                        + c1b2_ref[...])
        t2 = jax.nn.relu(b + t)
        out_ref[...] = (jnp.dot(t2, linw_ref[...],
                                preferred_element_type=jnp.float32)
                        + linb_ref[...])


def _row_spec(last):
    return pl.BlockSpec((_BLK, last), lambda i: (i, 0))


def _full_spec(shape):
    nd = len(shape)
    return pl.BlockSpec(shape, lambda i: (0,) * nd)


def _sc_spec(last):
    return pl.BlockSpec((NSC, _BLK, last), lambda i: (0, i, 0))


def kernel(x, edge_index, batch, W1, b1, W2, b2,
           c0w1, c0b1, c0w2, c0b2, c1w1, c1b1, c1w2, c1b2, lin_w, lin_b):
    src = edge_index[0]
    dst = edge_index[1]
    _deg_kernel, _agg_kernel = _sc_kernels()

    degp = _deg_kernel(dst).reshape(NSC, NPAD, 1)

    u1, dinv = pl.pallas_call(
        _tc1_body,
        grid=(_NBLK,),
        in_specs=[_row_spec(D), _full_spec((D, H)), _sc_spec(1)],
        out_specs=[_row_spec(H), _row_spec(1)],
        out_shape=[jax.ShapeDtypeStruct((N, H), jnp.float32),
                   jax.ShapeDtypeStruct((N, 1), jnp.float32)],
    )(x, W1, degp)

    s1 = _agg_kernel(u1, src, dst)

    u2 = pl.pallas_call(
        _tc2_body,
        grid=(_NBLK,),
        in_specs=[_sc_spec(H), _row_spec(H), _row_spec(1),
                  _full_spec((1, H)), _full_spec((H, H))],
        out_specs=_row_spec(H),
        out_shape=jax.ShapeDtypeStruct((N, H), jnp.float32),
    )(s1, u1, dinv, b1.reshape(1, H), W2)

    s2 = _agg_kernel(u2, src, dst)

    out = pl.pallas_call(
        _tc3_body,
        grid=(_NBLK,),
        in_specs=[_sc_spec(H), _row_spec(H), _row_spec(1),
                  _full_spec((1, H)), _row_spec(1),
                  _full_spec((H, H)), _full_spec((H, H)),
                  _full_spec((1, H)), _full_spec((1, H)),
                  _full_spec((H, H)), _full_spec((H, H)),
                  _full_spec((1, H)), _full_spec((1, H)),
                  _full_spec((H, NCLS)), _full_spec((1, NCLS))],
        out_specs=_full_spec((G, NCLS)),
        out_shape=jax.ShapeDtypeStruct((G, NCLS), jnp.float32),
        scratch_shapes=[pltpu.VMEM((G, H), jnp.float32),
                        pltpu.VMEM((G, 1), jnp.float32)],
    )(s2, u2, dinv, b2.reshape(1, H), batch.reshape(N, 1),
      c0w1[:, :, -1].T, c0w2[:, :, -1].T,
      c0b1.reshape(1, H), c0b2.reshape(1, H),
      c1w1[:, :, -1].T, c1w2[:, :, -1].T,
      c1b1.reshape(1, H), c1b2.reshape(1, H),
      lin_w, lin_b.reshape(1, NCLS))

    return out


# speedup vs baseline: 1.0059x; 1.0017x over previous
import functools

import jax
import jax.numpy as jnp
from jax import lax
from jax.experimental import pallas as pl
from jax.experimental.pallas import tpu as pltpu
from jax.experimental.pallas import tpu_sc as plsc

N = 10000
E = 320000
D = 128
H = 128
G = 256
NCLS = 10

NSC = 2
NTILE = 16
NW = NSC * NTILE
NPAD = 10240
RPT = NPAD // NTILE

EPW = E // NW

CHD = 80
NCHD = EPW // CHD

CH = 80
NCHUNK = EPW // CH


def _deg_body(dst_hbm, out_hbm, idx0, idx1, ones_v, stage_v, i0, i1, table):
    idx = (idx0, idx1)
    isem = (i0, i1)
    c = lax.axis_index("c")
    s = lax.axis_index("s")
    w = c * NTILE + s

    def _ones(i, _):
        ones_v[pl.ds(i * 16, 16)] = jnp.ones((16,), jnp.float32)
        return 0

    lax.fori_loop(0, CHD // 16, _ones, 0)

    def _zero(i, _):
        stage_v[pl.ds(i * 16, 16)] = jnp.zeros((16,), jnp.float32)
        return 0

    lax.fori_loop(0, RPT // 16, _zero, 0)
    pltpu.sync_copy(stage_v, table.at[pl.ds(s * RPT, RPT)])
    plsc.subcore_barrier()

    def _chunk(j, b, early, pref):
        if not early:
            pltpu.make_async_copy(dst_hbm.at[pl.ds(0, CHD)], idx[b],
                                  isem[b]).wait()
        pltpu.sync_copy(ones_v, table.at[idx[b]], add=True)
        if pref:
            pltpu.async_copy(dst_hbm.at[pl.ds(w * EPW + (j + 2) * CHD,
                                              CHD)], idx[b], isem[b])

    for j in range(2):
        pltpu.sync_copy(dst_hbm.at[pl.ds(w * EPW + j * CHD, CHD)], idx[j])
    _chunk(0, 0, True, True)
    _chunk(1, 1, True, True)

    def _body(g, _):
        j0 = 2 * g + 2
        _chunk(j0, 0, False, True)
        _chunk(j0 + 1, 1, False, True)
        return 0

    lax.fori_loop(0, (NCHD - 5) // 2, _body, 0)
    _chunk(NCHD - 3, 0, False, True)
    _chunk(NCHD - 2, 1, False, False)
    _chunk(NCHD - 1, 0, False, False)
    plsc.subcore_barrier()
    pltpu.sync_copy(table.at[pl.ds(s * RPT, RPT)],
                    out_hbm.at[c, pl.ds(s * RPT, RPT)])


def _agg_body(u_hbm, src_hbm, dst_hbm, out_hbm, sidx0, sidx1, didx0, didx1,
              rows0, rows1, gs0, gs1, iss0, iss1, isd0, isd1, ss0, ss1,
              table):
    rows = (rows0, rows1)
    sidx = (sidx0, sidx1)
    didx = (didx0, didx1)
    gsem = (gs0, gs1)
    isem_s = (iss0, iss1)
    isem_d = (isd0, isd1)
    ssem = (ss0, ss1)
    c = lax.axis_index("c")
    s = lax.axis_index("s")
    w = c * NTILE + s

    def _zero(i, _):
        rows0[i // 8, pl.ds((i % 8) * 16, 16)] = jnp.zeros((16,),
                                                           jnp.float32)
        return 0

    lax.fori_loop(0, CH * (H // 16), _zero, 0)

    def _ztab(k, _):
        pltpu.sync_copy(rows0, table.at[pl.ds(s * RPT + k * CH, CH)])
        return 0

    lax.fori_loop(0, RPT // CH, _ztab, 0)
    plsc.subcore_barrier()

    def _wait_g(b):
        pltpu.make_async_copy(u_hbm.at[sidx[b]], rows[b], gsem[b]).wait()

    def _wait_is(p):
        pltpu.make_async_copy(src_hbm.at[pl.ds(0, CH)], sidx[p],
                              isem_s[p]).wait()

    def _wait_id(p):
        pltpu.make_async_copy(dst_hbm.at[pl.ds(0, CH)], didx[p],
                              isem_d[p]).wait()

    def _chunk(j, b, fi, sec, last, pref):
        _wait_g(b)
        if not last:
            if not fi:
                _wait_is(1 - b)
                pltpu.async_copy(
                    dst_hbm.at[pl.ds(w * EPW + (j + 1) * CH, CH)],
                    didx[1 - b], isem_d[1 - b])
            pltpu.async_copy(u_hbm.at[sidx[1 - b]], rows[1 - b],
                             gsem[1 - b])
        if not (fi or sec):
            _wait_id(b)
        pltpu.sync_copy(rows[b], table.at[didx[b]], add=True)
        if pref:
            pltpu.async_copy(
                src_hbm.at[pl.ds(w * EPW + (j + 2) * CH, CH)],
                sidx[b], isem_s[b])

    for j in range(2):
        pltpu.sync_copy(src_hbm.at[pl.ds(w * EPW + j * CH, CH)], sidx[j])
        pltpu.sync_copy(dst_hbm.at[pl.ds(w * EPW + j * CH, CH)], didx[j])
    pltpu.async_copy(u_hbm.at[sidx0], rows0, gsem[0])
    _chunk(0, 0, True, False, False, True)
    _chunk(1, 1, False, True, False, True)

    def _body(g, _):
        j0 = 2 * g + 2
        _chunk(j0, 0, False, False, False, True)
        _chunk(j0 + 1, 1, False, False, False, True)
        return 0

    lax.fori_loop(0, (NCHUNK - 5) // 2, _body, 0)
    _chunk(NCHUNK - 3, 0, False, False, False, True)
    _chunk(NCHUNK - 2, 1, False, False, False, False)
    _chunk(NCHUNK - 1, 0, False, False, True, False)
    plsc.subcore_barrier()
    pltpu.sync_copy(table.at[pl.ds(s * RPT, RPT)],
                    out_hbm.at[c, pl.ds(s * RPT, RPT)])


@functools.cache
def _sc_kernels():
    mesh = plsc.VectorSubcoreMesh(core_axis_name="c", subcore_axis_name="s")
    deg = pl.kernel(
        _deg_body,
        out_type=jax.ShapeDtypeStruct((NSC, NPAD), jnp.float32),
        mesh=mesh,
        scratch_types=[
            pltpu.VMEM((CHD,), jnp.int32),
            pltpu.VMEM((CHD,), jnp.int32),
            pltpu.VMEM((CHD,), jnp.float32),
            pltpu.VMEM((RPT,), jnp.float32),
            pltpu.SemaphoreType.DMA,
            pltpu.SemaphoreType.DMA,
            pltpu.VMEM_SHARED((NPAD,), jnp.float32),
        ],
    )
    agg = pl.kernel(
        _agg_body,
        out_type=jax.ShapeDtypeStruct((NSC, NPAD, H), jnp.float32),
        mesh=mesh,
        scratch_types=[
            pltpu.VMEM((CH,), jnp.int32),
            pltpu.VMEM((CH,), jnp.int32),
            pltpu.VMEM((CH,), jnp.int32),
            pltpu.VMEM((CH,), jnp.int32),
            pltpu.VMEM((CH, H), jnp.float32),
            pltpu.VMEM((CH, H), jnp.float32),
            pltpu.SemaphoreType.DMA,
            pltpu.SemaphoreType.DMA,
            pltpu.SemaphoreType.DMA,
            pltpu.SemaphoreType.DMA,
            pltpu.SemaphoreType.DMA,
            pltpu.SemaphoreType.DMA,
            pltpu.SemaphoreType.DMA,
            pltpu.SemaphoreType.DMA,
            pltpu.VMEM_SHARED((NPAD, H), jnp.float32),
        ],
    )
    return deg, agg


_BLK = 2000
_NBLK = N // _BLK


def _tc1_body(x_ref, w1_ref, deg_ref, u_ref, dinv_ref):
    deg = deg_ref[0] + deg_ref[1] + 1.0
    dinv = lax.rsqrt(deg)
    dinv_ref[...] = dinv
    u_ref[...] = jnp.dot(x_ref[...], w1_ref[...],
                         preferred_element_type=jnp.float32) * dinv


def _tc2_body(s_ref, u1_ref, dinv_ref, b1_ref, w2_ref, u2_ref):
    dinv = dinv_ref[...]
    h = jax.nn.relu(dinv * (s_ref[0] + s_ref[1] + u1_ref[...]) + b1_ref[...])
    u2_ref[...] = jnp.dot(h, w2_ref[...],
                          preferred_element_type=jnp.float32) * dinv


def _tc3_body(s_ref, u2_ref, dinv_ref, b2_ref, batch_ref,
              t0a_ref, t0b_ref, c0b1_ref, c0b2_ref,
              t1a_ref, t1b_ref, c1b1_ref, c1b2_ref,
              linw_ref, linb_ref, out_ref, sums, cnt):
    i = pl.program_id(0)

    @pl.when(i == 0)
    def _():
        sums[...] = jnp.zeros_like(sums)
        cnt[...] = jnp.zeros_like(cnt)

    h = jax.nn.relu(dinv_ref[...] * (s_ref[0] + s_ref[1] + u2_ref[...])
                    + b2_ref[...])
    gids = lax.broadcasted_iota(jnp.int32, (_BLK, G), 1)
    onehot = (batch_ref[...] == gids).astype(jnp.float32)
    dn = (((0,), (0,)), ((), ()))
    sums[...] += lax.dot_general(onehot, h, dn,
                                 preferred_element_type=jnp.float32)
    cnt[...] += lax.dot_general(onehot, jnp.ones((_BLK, 1), jnp.float32), dn,
                                preferred_element_type=jnp.float32)

    @pl.when(i == _NBLK - 1)
    def _():
        pooled = sums[...] / jnp.maximum(cnt[...], 1.0)
        a = jax.nn.relu(jnp.dot(pooled, t0a_ref[...],
                                preferred_element_type=jnp.float32)
                        + c0b1_ref[...])
        a = jax.nn.relu(jnp.dot(a, t0b_ref[...],
                                preferred_element_type=jnp.float32)
                        + c0b2_ref[...])
        t = jax.nn.relu(a + pooled)
        b = jax.nn.relu(jnp.dot(t, t1a_ref[...],
                                preferred_element_type=jnp.float32)
                        + c1b1_ref[...])
        b = jax.nn.relu(jnp.dot(b, t1b_ref[...],
                                preferred_element_type=jnp.float32)
                        + c1b2_ref[...])
        t2 = jax.nn.relu(b + t)
        out_ref[...] = (jnp.dot(t2, linw_ref[...],
                                preferred_element_type=jnp.float32)
                        + linb_ref[...])


def _row_spec(last):
    return pl.BlockSpec((_BLK, last), lambda i: (i, 0))


def _full_spec(shape):
    nd = len(shape)
    return pl.BlockSpec(shape, lambda i: (0,) * nd)


def _sc_spec(last):
    return pl.BlockSpec((NSC, _BLK, last), lambda i: (0, i, 0))


def kernel(x, edge_index, batch, W1, b1, W2, b2,
           c0w1, c0b1, c0w2, c0b2, c1w1, c1b1, c1w2, c1b2, lin_w, lin_b):
    src = edge_index[0]
    dst = edge_index[1]
    _deg_kernel, _agg_kernel = _sc_kernels()

    degp = _deg_kernel(dst).reshape(NSC, NPAD, 1)

    u1, dinv = pl.pallas_call(
        _tc1_body,
        grid=(_NBLK,),
        in_specs=[_row_spec(D), _full_spec((D, H)), _sc_spec(1)],
        out_specs=[_row_spec(H), _row_spec(1)],
        out_shape=[jax.ShapeDtypeStruct((N, H), jnp.float32),
                   jax.ShapeDtypeStruct((N, 1), jnp.float32)],
    )(x, W1, degp)

    s1 = _agg_kernel(u1, src, dst)

    u2 = pl.pallas_call(
        _tc2_body,
        grid=(_NBLK,),
        in_specs=[_sc_spec(H), _row_spec(H), _row_spec(1),
                  _full_spec((1, H)), _full_spec((H, H))],
        out_specs=_row_spec(H),
        out_shape=jax.ShapeDtypeStruct((N, H), jnp.float32),
    )(s1, u1, dinv, b1.reshape(1, H), W2)

    s2 = _agg_kernel(u2, src, dst)

    out = pl.pallas_call(
        _tc3_body,
        grid=(_NBLK,),
        in_specs=[_sc_spec(H), _row_spec(H), _row_spec(1),
                  _full_spec((1, H)), _row_spec(1),
                  _full_spec((H, H)), _full_spec((H, H)),
                  _full_spec((1, H)), _full_spec((1, H)),
                  _full_spec((H, H)), _full_spec((H, H)),
                  _full_spec((1, H)), _full_spec((1, H)),
                  _full_spec((H, NCLS)), _full_spec((1, NCLS))],
        out_specs=_full_spec((G, NCLS)),
        out_shape=jax.ShapeDtypeStruct((G, NCLS), jnp.float32),
        scratch_shapes=[pltpu.VMEM((G, H), jnp.float32),
                        pltpu.VMEM((G, 1), jnp.float32)],
    )(s2, u2, dinv, b2.reshape(1, H), batch.reshape(N, 1),
      c0w1[:, :, -1].T, c0w2[:, :, -1].T,
      c0b1.reshape(1, H), c0b2.reshape(1, H),
      c1w1[:, :, -1].T, c1w2[:, :, -1].T,
      c1b1.reshape(1, H), c1b2.reshape(1, H),
      lin_w, lin_b.reshape(1, NCLS))

    return out
